# baseline (speedup 1.0000x reference)
import functools

import jax
import jax.numpy as jnp
from jax import lax
from jax.experimental import pallas as pl
from jax.experimental.pallas import tpu as pltpu
from jax.experimental.pallas import tpu_sc as plsc

N = 10000
E = 320000
D = 128
ED = 16

NC = 2
NS = 16
NW = NC * NS
CH = 128
NCHUNK = 79
EPW = NCHUNK * CH
E_PAD = NW * EPW
NP = 10240
RPT = NP // NS
K0 = 112
K1 = 48
TOT_CH = NS * (K0 + K1)
TOT_PAD = NS * K1 + NS * K0 + K0

_f32 = jnp.float32


_sc_mesh = plsc.VectorSubcoreMesh(core_axis_name="c", subcore_axis_name="s")


def _scatter_pass(load, drain, dst_v, r0v, r1v, s0, s1, accx):
    load(0, r0v, s0)

    @pl.loop(0, (NCHUNK - 1) // 2)
    def _(i):
        a = 2 * i
        load(a + 1, r1v, s1)
        drain(r0v, s0)
        pltpu.sync_copy(r0v, accx.at[dst_v.at[a]], add=True)
        load(a + 2, r0v, s0)
        drain(r1v, s1)
        pltpu.sync_copy(r1v, accx.at[dst_v.at[a + 1]], add=True)

    drain(r0v, s0)
    pltpu.sync_copy(r0v, accx.at[dst_v.at[NCHUNK - 1]], add=True)


@functools.partial(
    pl.kernel,
    out_type=jax.ShapeDtypeStruct((NC, NP, D), _f32),
    mesh=_sc_mesh,
    scratch_types=[
        pltpu.VMEM((2, CH), jnp.int32),
        pltpu.VMEM((K0, CH), jnp.int32),
        pltpu.VMEM((CH, D), _f32),
        pltpu.VMEM((CH, D), _f32),
        pltpu.VMEM_SHARED((NP, D), _f32),
        pltpu.SemaphoreType.DMA,
        pltpu.SemaphoreType.DMA,
        pltpu.SemaphoreType.DMA,
        pltpu.SemaphoreType.DMA,
    ],
)
def _sc_layer(x_hbm, srcs_hbm, dsts_hbm, zx_hbm, outx_hbm,
              src_i, dst_v, r0v, r1v, accx, g0, g1, i0, i1):
    cid = lax.axis_index("c")
    sid = lax.axis_index("s")
    r0 = sid * RPT
    S = jnp.where(cid == 0, sid * K1, NS * K1 + sid * K0)
    K = jnp.where(cid == 0, K1, K0)
    pltpu.sync_copy(zx_hbm.at[pl.ds(r0, RPT)], accx.at[pl.ds(r0, RPT)])
    pltpu.sync_copy(dsts_hbm.at[pl.ds(S, K0)], dst_v)
    plsc.subcore_barrier()

    def idx_load(j, b, sem):
        pltpu.async_copy(srcs_hbm.at[S + j], src_i.at[b], sem)

    def idx_wait(b, sem):
        pltpu.make_async_copy(srcs_hbm.at[0], src_i.at[b], sem).wait()

    def gather(b, buf, sem):
        pltpu.async_copy(x_hbm.at[src_i.at[b]], buf, sem)

    def gather_wait(buf, sem):
        pltpu.make_async_copy(x_hbm.at[pl.ds(0, CH)], buf, sem).wait()

    idx_load(0, 0, i0)
    idx_wait(0, i0)
    gather(0, r0v, g0)
    idx_load(1, 1, i1)

    @pl.loop(0, K // 2)
    def _(i):
        a = 2 * i
        idx_wait(1, i1)
        gather(1, r1v, g1)
        gather_wait(r0v, g0)
        pltpu.sync_copy(r0v, accx.at[dst_v.at[a]], add=True)
        idx_load(jnp.minimum(a + 2, K - 1), 0, i0)
        idx_wait(0, i0)
        gather(0, r0v, g0)
        gather_wait(r1v, g1)
        pltpu.sync_copy(r1v, accx.at[dst_v.at[a + 1]], add=True)
        idx_load(jnp.minimum(a + 3, K - 1), 1, i1)

    idx_wait(1, i1)
    gather_wait(r0v, g0)

    plsc.subcore_barrier()
    pltpu.sync_copy(accx.at[pl.ds(r0, RPT)], outx_hbm.at[cid, pl.ds(r0, RPT)])


PK = CH // 8


@functools.partial(
    pl.kernel,
    out_type=jax.ShapeDtypeStruct((NC, NP, D), _f32),
    mesh=_sc_mesh,
    scratch_types=[
        pltpu.VMEM((NCHUNK, CH), jnp.int32),
        pltpu.VMEM((PK, D), _f32),
        pltpu.VMEM((PK, D), _f32),
        pltpu.VMEM((CH, D), _f32),
        pltpu.VMEM_SHARED((NP, D), _f32),
        pltpu.SemaphoreType.DMA,
        pltpu.SemaphoreType.DMA,
    ],
)
def _sc_ea(eap_hbm, dsts_hbm, zx_hbm, outea_hbm,
           dst_v, p0v, p1v, stg, accea, s0, s1):
    cid = lax.axis_index("c")
    sid = lax.axis_index("s")
    wid = cid * NS + sid
    r0 = sid * RPT
    pltpu.sync_copy(zx_hbm.at[pl.ds(r0, RPT)], accea.at[pl.ds(r0, RPT)])
    pltpu.sync_copy(dsts_hbm.at[wid], dst_v)

    zv = jnp.zeros((16,), _f32)
    onehot = jnp.where(lax.broadcasted_iota(jnp.int32, (16,), 0) == 0, 1.0,
                       0.0).astype(_f32)

    @pl.loop(0, CH)
    def _(r):
        stg[r, pl.ds(ED, 16)] = onehot
        for c in range(2, 8):
            stg[r, pl.ds(c * 16, 16)] = zv

    plsc.subcore_barrier()

    def load(j, buf, sem):
        pltpu.async_copy(eap_hbm.at[wid, j], buf, sem)

    def drain(buf, sem):
        pltpu.make_async_copy(eap_hbm.at[0, 0], buf, sem).wait()

    def unpack_scatter(j, buf):
        for e in range(CH):
            stg[e, pl.ds(0, ED)] = buf[e // 8, pl.ds(ED * (e % 8), ED)]
        pltpu.sync_copy(stg, accea.at[dst_v.at[j]], add=True)

    load(0, p0v, s0)

    @pl.loop(0, (NCHUNK - 1) // 2)
    def _(i):
        a = 2 * i
        load(a + 1, p1v, s1)
        drain(p0v, s0)
        unpack_scatter(a, p0v)
        load(a + 2, p0v, s0)
        drain(p1v, s1)
        unpack_scatter(a + 1, p1v)

    drain(p0v, s0)
    unpack_scatter(NCHUNK - 1, p0v)

    plsc.subcore_barrier()
    pltpu.sync_copy(accea.at[pl.ds(r0, RPT)],
                    outea_hbm.at[cid, pl.ds(r0, RPT)])


BR = 512


def _tc_layer_body(sx_ref, sea_ref, x_ref, We_ref, be_ref, Wl_ref, bl_ref,
                   Wr_ref, o_ref):
    sx = sx_ref[0] + sx_ref[1]
    sea = sea_ref[0] + sea_ref[1]
    cnt = sea[:, ED:ED + 1]
    inv = 1.0 / jnp.maximum(cnt, 1.0)
    has = jnp.where(cnt > 0.0, 1.0, 0.0)
    hi = jax.lax.Precision.HIGHEST
    agg = (sx * inv
           + jnp.dot(sea[:, :ED] * inv, We_ref[...],
                     preferred_element_type=_f32, precision=hi)
           + be_ref[...] * has)
    out = (jnp.dot(agg, Wl_ref[...], preferred_element_type=_f32, precision=hi)
           + bl_ref[...]
           + jnp.dot(x_ref[...], Wr_ref[...],
                     preferred_element_type=_f32, precision=hi))
    o_ref[...] = jnp.maximum(out, 0.0)


def _tc_layer(sx, sea, x, We, be, Wl, bl, Wr):
    grid = (NP // BR,)
    return pl.pallas_call(
        _tc_layer_body,
        grid=grid,
        in_specs=[
            pl.BlockSpec((NC, BR, D), lambda i: (0, i, 0)),
            pl.BlockSpec((NC, BR, D), lambda i: (0, i, 0)),
            pl.BlockSpec((BR, D), lambda i: (i, 0)),
            pl.BlockSpec((ED, D), lambda i: (0, 0)),
            pl.BlockSpec((1, D), lambda i: (0, 0)),
            pl.BlockSpec((D, D), lambda i: (0, 0)),
            pl.BlockSpec((1, D), lambda i: (0, 0)),
            pl.BlockSpec((D, D), lambda i: (0, 0)),
        ],
        out_specs=pl.BlockSpec((BR, D), lambda i: (i, 0)),
        out_shape=jax.ShapeDtypeStruct((NP, D), _f32),
    )(sx, sea, x, We, be.reshape(1, D), Wl, bl.reshape(1, D), Wr)


def kernel(x, edge_index, edge_attr, Wl0, bl0, Wr0, We0, be0,
           Wl1, bl1, Wr1, We1, be1):
    src = edge_index[0]
    dst = edge_index[1]
    pad = E_PAD - E
    padf = TOT_PAD * CH - E
    pad_dstf = N + jnp.arange(padf, dtype=jnp.int32) % (NP - N)
    srcs = jnp.concatenate([src, jnp.zeros((padf,), jnp.int32)])
    srcs = srcs.reshape(TOT_PAD, CH)
    dst_pf = jnp.concatenate([dst, pad_dstf])
    dstsf = dst_pf.reshape(TOT_PAD, CH)
    dsts = dst_pf[:E_PAD].reshape(NW, NCHUNK, CH)
    eap = edge_attr.reshape(E // 8, D)
    eap = jnp.concatenate([eap, jnp.zeros((pad // 8, D), _f32)], axis=0)
    eap = eap.reshape(NW, NCHUNK, PK, D)
    x_pad = jnp.concatenate([x, jnp.zeros((NP - N, D), _f32)], axis=0)
    zx = jnp.zeros((NP, D), _f32)

    sea = _sc_ea(eap, dsts, zx)
    sx0 = _sc_layer(x_pad, srcs, dstsf, zx)
    x1 = _tc_layer(sx0, sea, x_pad, We0, be0, Wl0, bl0, Wr0)
    sx1 = _sc_layer(x1, srcs, dstsf, zx)
    x2 = _tc_layer(sx1, sea, x1, We1, be1, Wl1, bl1, Wr1)
    return x2[:N]


# speedup vs baseline: 1.0191x; 1.0191x over previous
import functools

import jax
import jax.numpy as jnp
from jax import lax
from jax.experimental import pallas as pl
from jax.experimental.pallas import tpu as pltpu
from jax.experimental.pallas import tpu_sc as plsc

N = 10000
E = 320000
D = 128
ED = 16

NC = 2
NS = 16
NW = NC * NS
CH = 128
NCHUNK = 79
EPW = NCHUNK * CH
E_PAD = NW * EPW
NP = 10240
RPT = NP // NS
KX = 80
TOT_CH = NW * KX
TOT_PAD = TOT_CH

_f32 = jnp.float32


_sc_mesh = plsc.VectorSubcoreMesh(core_axis_name="c", subcore_axis_name="s")


def _scatter_pass(load, drain, dst_v, r0v, r1v, s0, s1, accx):
    load(0, r0v, s0)

    @pl.loop(0, (NCHUNK - 1) // 2)
    def _(i):
        a = 2 * i
        load(a + 1, r1v, s1)
        drain(r0v, s0)
        pltpu.sync_copy(r0v, accx.at[dst_v.at[a]], add=True)
        load(a + 2, r0v, s0)
        drain(r1v, s1)
        pltpu.sync_copy(r1v, accx.at[dst_v.at[a + 1]], add=True)

    drain(r0v, s0)
    pltpu.sync_copy(r0v, accx.at[dst_v.at[NCHUNK - 1]], add=True)


@functools.partial(
    pl.kernel,
    out_type=jax.ShapeDtypeStruct((NC, NP, D), _f32),
    mesh=_sc_mesh,
    scratch_types=[
        pltpu.VMEM((2, CH), jnp.int32),
        pltpu.VMEM((KX, CH), jnp.int32),
        pltpu.VMEM((CH, D), _f32),
        pltpu.VMEM((CH, D), _f32),
        pltpu.VMEM_SHARED((NP, D), _f32),
        pltpu.SemaphoreType.DMA,
        pltpu.SemaphoreType.DMA,
        pltpu.SemaphoreType.DMA,
        pltpu.SemaphoreType.DMA,
    ],
)
def _sc_layer(x_hbm, srcs_hbm, dsts_hbm, zx_hbm, outx_hbm,
              src_i, dst_v, r0v, r1v, accx, g0, g1, i0, i1):
    cid = lax.axis_index("c")
    sid = lax.axis_index("s")
    r0 = sid * RPT
    S = (cid * NS + sid) * KX
    K = KX
    pltpu.sync_copy(zx_hbm.at[pl.ds(r0, RPT)], accx.at[pl.ds(r0, RPT)])
    pltpu.sync_copy(dsts_hbm.at[pl.ds(S, KX)], dst_v)
    plsc.subcore_barrier()

    def idx_load(j, b, sem):
        pltpu.async_copy(srcs_hbm.at[S + j], src_i.at[b], sem)

    def idx_wait(b, sem):
        pltpu.make_async_copy(srcs_hbm.at[0], src_i.at[b], sem).wait()

    def gather(b, buf, sem):
        pltpu.async_copy(x_hbm.at[src_i.at[b]], buf, sem)

    def gather_wait(buf, sem):
        pltpu.make_async_copy(x_hbm.at[pl.ds(0, CH)], buf, sem).wait()

    idx_load(0, 0, i0)
    idx_wait(0, i0)
    gather(0, r0v, g0)
    idx_load(1, 1, i1)

    @pl.loop(0, K // 2)
    def _(i):
        a = 2 * i
        idx_wait(1, i1)
        gather(1, r1v, g1)
        gather_wait(r0v, g0)
        pltpu.sync_copy(r0v, accx.at[dst_v.at[a]], add=True)
        idx_load(jnp.minimum(a + 2, K - 1), 0, i0)
        idx_wait(0, i0)
        gather(0, r0v, g0)
        gather_wait(r1v, g1)
        pltpu.sync_copy(r1v, accx.at[dst_v.at[a + 1]], add=True)
        idx_load(jnp.minimum(a + 3, K - 1), 1, i1)

    idx_wait(1, i1)
    gather_wait(r0v, g0)

    plsc.subcore_barrier()
    pltpu.sync_copy(accx.at[pl.ds(r0, RPT)], outx_hbm.at[cid, pl.ds(r0, RPT)])


PK = CH // 8


@functools.partial(
    pl.kernel,
    out_type=jax.ShapeDtypeStruct((NC, NP, D), _f32),
    mesh=_sc_mesh,
    scratch_types=[
        pltpu.VMEM((NCHUNK, CH), jnp.int32),
        pltpu.VMEM((PK, D), _f32),
        pltpu.VMEM((PK, D), _f32),
        pltpu.VMEM((CH, D), _f32),
        pltpu.VMEM_SHARED((NP, D), _f32),
        pltpu.SemaphoreType.DMA,
        pltpu.SemaphoreType.DMA,
    ],
)
def _sc_ea(eap_hbm, dsts_hbm, zx_hbm, outea_hbm,
           dst_v, p0v, p1v, stg, accea, s0, s1):
    cid = lax.axis_index("c")
    sid = lax.axis_index("s")
    wid = cid * NS + sid
    r0 = sid * RPT
    pltpu.sync_copy(zx_hbm.at[pl.ds(r0, RPT)], accea.at[pl.ds(r0, RPT)])
    pltpu.sync_copy(dsts_hbm.at[wid], dst_v)

    zv = jnp.zeros((16,), _f32)
    onehot = jnp.where(lax.broadcasted_iota(jnp.int32, (16,), 0) == 0, 1.0,
                       0.0).astype(_f32)

    @pl.loop(0, CH)
    def _(r):
        stg[r, pl.ds(ED, 16)] = onehot
        for c in range(2, 8):
            stg[r, pl.ds(c * 16, 16)] = zv

    plsc.subcore_barrier()

    def load(j, buf, sem):
        pltpu.async_copy(eap_hbm.at[wid, j], buf, sem)

    def drain(buf, sem):
        pltpu.make_async_copy(eap_hbm.at[0, 0], buf, sem).wait()

    def unpack_scatter(j, buf):
        for e in range(CH):
            stg[e, pl.ds(0, ED)] = buf[e // 8, pl.ds(ED * (e % 8), ED)]
        pltpu.sync_copy(stg, accea.at[dst_v.at[j]], add=True)

    load(0, p0v, s0)

    @pl.loop(0, (NCHUNK - 1) // 2)
    def _(i):
        a = 2 * i
        load(a + 1, p1v, s1)
        drain(p0v, s0)
        unpack_scatter(a, p0v)
        load(a + 2, p0v, s0)
        drain(p1v, s1)
        unpack_scatter(a + 1, p1v)

    drain(p0v, s0)
    unpack_scatter(NCHUNK - 1, p0v)

    plsc.subcore_barrier()
    pltpu.sync_copy(accea.at[pl.ds(r0, RPT)],
                    outea_hbm.at[cid, pl.ds(r0, RPT)])


BR = 512


def _tc_layer_body(sx_ref, sea_ref, x_ref, We_ref, be_ref, Wl_ref, bl_ref,
                   Wr_ref, o_ref):
    sx = sx_ref[0] + sx_ref[1]
    sea = sea_ref[0] + sea_ref[1]
    cnt = sea[:, ED:ED + 1]
    inv = 1.0 / jnp.maximum(cnt, 1.0)
    has = jnp.where(cnt > 0.0, 1.0, 0.0)
    hi = jax.lax.Precision.HIGHEST
    agg = (sx * inv
           + jnp.dot(sea[:, :ED] * inv, We_ref[...],
                     preferred_element_type=_f32, precision=hi)
           + be_ref[...] * has)
    out = (jnp.dot(agg, Wl_ref[...], preferred_element_type=_f32, precision=hi)
           + bl_ref[...]
           + jnp.dot(x_ref[...], Wr_ref[...],
                     preferred_element_type=_f32, precision=hi))
    o_ref[...] = jnp.maximum(out, 0.0)


def _tc_layer(sx, sea, x, We, be, Wl, bl, Wr):
    grid = (NP // BR,)
    return pl.pallas_call(
        _tc_layer_body,
        grid=grid,
        in_specs=[
            pl.BlockSpec((NC, BR, D), lambda i: (0, i, 0)),
            pl.BlockSpec((NC, BR, D), lambda i: (0, i, 0)),
            pl.BlockSpec((BR, D), lambda i: (i, 0)),
            pl.BlockSpec((ED, D), lambda i: (0, 0)),
            pl.BlockSpec((1, D), lambda i: (0, 0)),
            pl.BlockSpec((D, D), lambda i: (0, 0)),
            pl.BlockSpec((1, D), lambda i: (0, 0)),
            pl.BlockSpec((D, D), lambda i: (0, 0)),
        ],
        out_specs=pl.BlockSpec((BR, D), lambda i: (i, 0)),
        out_shape=jax.ShapeDtypeStruct((NP, D), _f32),
    )(sx, sea, x, We, be.reshape(1, D), Wl, bl.reshape(1, D), Wr)


def kernel(x, edge_index, edge_attr, Wl0, bl0, Wr0, We0, be0,
           Wl1, bl1, Wr1, We1, be1):
    src = edge_index[0]
    dst = edge_index[1]
    pad = E_PAD - E
    padf = TOT_PAD * CH - E
    pad_dstf = N + jnp.arange(padf, dtype=jnp.int32) % (NP - N)
    srcs = jnp.concatenate([src, jnp.zeros((padf,), jnp.int32)])
    srcs = srcs.reshape(TOT_PAD, CH)
    dst_pf = jnp.concatenate([dst, pad_dstf])
    dstsf = dst_pf.reshape(TOT_PAD, CH)
    dsts = dst_pf[:E_PAD].reshape(NW, NCHUNK, CH)
    eap = edge_attr.reshape(E // 8, D)
    eap = jnp.concatenate([eap, jnp.zeros((pad // 8, D), _f32)], axis=0)
    eap = eap.reshape(NW, NCHUNK, PK, D)
    x_pad = jnp.concatenate([x, jnp.zeros((NP - N, D), _f32)], axis=0)
    zx = jnp.zeros((NP, D), _f32)

    sea = _sc_ea(eap, dsts, zx)
    sx0 = _sc_layer(x_pad, srcs, dstsf, zx)
    x1 = _tc_layer(sx0, sea, x_pad, We0, be0, Wl0, bl0, Wr0)
    sx1 = _sc_layer(x1, srcs, dstsf, zx)
    x2 = _tc_layer(sx1, sea, x1, We1, be1, Wl1, bl1, Wr1)
    return x2[:N]


# speedup vs baseline: 1.5521x; 1.5230x over previous
import functools

import jax
import jax.numpy as jnp
from jax import lax
from jax.experimental import pallas as pl
from jax.experimental.pallas import tpu as pltpu
from jax.experimental.pallas import tpu_sc as plsc

N = 10000
E = 320000
D = 128
ED = 16

NC = 2
NS = 16
NW = NC * NS
CH = 128
NCHUNK = 79
EPW = NCHUNK * CH
E_PAD = NW * EPW
NP = 10240
RPT = NP // NS
PK = CH // 8

_f32 = jnp.float32

_sc_mesh = plsc.VectorSubcoreMesh(core_axis_name="c", subcore_axis_name="s")


@functools.partial(
    pl.kernel,
    out_type=jax.ShapeDtypeStruct((NC, NP, D), _f32),
    mesh=_sc_mesh,
    scratch_types=[
        pltpu.VMEM((2, CH), jnp.int32),
        pltpu.VMEM((NCHUNK, CH), jnp.int32),
        pltpu.VMEM((CH, D), _f32),
        pltpu.VMEM((CH, D), _f32),
        pltpu.VMEM_SHARED((NP, D), _f32),
        pltpu.SemaphoreType.DMA,
        pltpu.SemaphoreType.DMA,
        pltpu.SemaphoreType.DMA,
        pltpu.SemaphoreType.DMA,
    ],
)
def _sc_layer(x_hbm, srcs_hbm, dsts_hbm, zx_hbm, outx_hbm,
              src_i, dst_v, r0v, r1v, accx, g0, g1, i0, i1):
    cid = lax.axis_index("c")
    sid = lax.axis_index("s")
    wid = cid * NS + sid
    r0 = sid * RPT
    pltpu.sync_copy(zx_hbm.at[pl.ds(r0, RPT)], accx.at[pl.ds(r0, RPT)])
    pltpu.sync_copy(dsts_hbm.at[wid], dst_v)
    plsc.subcore_barrier()

    def idx_load(j, b, sem):
        pltpu.async_copy(srcs_hbm.at[wid, j], src_i.at[b], sem)

    def idx_wait(b, sem):
        pltpu.make_async_copy(srcs_hbm.at[wid, 0], src_i.at[b], sem).wait()

    def gather(b, buf, sem):
        pltpu.async_copy(x_hbm.at[src_i.at[b]], buf, sem)

    def gather_wait(buf, sem):
        pltpu.make_async_copy(x_hbm.at[pl.ds(0, CH)], buf, sem).wait()

    idx_load(0, 0, i0)
    idx_wait(0, i0)
    gather(0, r0v, g0)
    idx_load(1, 1, i1)

    @pl.loop(0, (NCHUNK - 1) // 2)
    def _(i):
        a = 2 * i
        idx_wait(1, i1)
        gather(1, r1v, g1)
        gather_wait(r0v, g0)
        pltpu.sync_copy(r0v, accx.at[dst_v.at[a]], add=True)
        idx_load(a + 2, 0, i0)
        idx_wait(0, i0)
        gather(0, r0v, g0)
        gather_wait(r1v, g1)
        pltpu.sync_copy(r1v, accx.at[dst_v.at[a + 1]], add=True)
        idx_load(jnp.minimum(a + 3, NCHUNK - 1), 1, i1)

    idx_wait(1, i1)
    gather_wait(r0v, g0)
    pltpu.sync_copy(r0v, accx.at[dst_v.at[NCHUNK - 1]], add=True)

    plsc.subcore_barrier()
    pltpu.sync_copy(accx.at[pl.ds(r0, RPT)], outx_hbm.at[cid, pl.ds(r0, RPT)])


@functools.partial(
    pl.kernel,
    out_type=jax.ShapeDtypeStruct((NC, NP, D), _f32),
    mesh=_sc_mesh,
    scratch_types=[
        pltpu.VMEM((NCHUNK, CH), jnp.int32),
        pltpu.VMEM((PK, D), _f32),
        pltpu.VMEM((PK, D), _f32),
        pltpu.VMEM((CH, D), _f32),
        pltpu.VMEM_SHARED((NP, D), _f32),
        pltpu.SemaphoreType.DMA,
        pltpu.SemaphoreType.DMA,
    ],
)
def _sc_ea(eap_hbm, dsts_hbm, zx_hbm, outea_hbm,
           dst_v, p0v, p1v, stg, accea, s0, s1):
    cid = lax.axis_index("c")
    sid = lax.axis_index("s")
    wid = cid * NS + sid
    r0 = sid * RPT
    pltpu.sync_copy(zx_hbm.at[pl.ds(r0, RPT)], accea.at[pl.ds(r0, RPT)])
    pltpu.sync_copy(dsts_hbm.at[wid], dst_v)

    zv = jnp.zeros((16,), _f32)
    onehot = jnp.where(lax.broadcasted_iota(jnp.int32, (16,), 0) == 0, 1.0,
                       0.0).astype(_f32)

    @pl.loop(0, CH)
    def _(r):
        stg[r, pl.ds(ED, 16)] = onehot
        for c in range(2, 8):
            stg[r, pl.ds(c * 16, 16)] = zv

    plsc.subcore_barrier()

    def load(j, buf, sem):
        pltpu.async_copy(eap_hbm.at[wid, j], buf, sem)

    def drain(buf, sem):
        pltpu.make_async_copy(eap_hbm.at[0, 0], buf, sem).wait()

    def unpack_scatter(j, buf):
        for e in range(CH):
            stg[e, pl.ds(0, ED)] = buf[e // 8, pl.ds(ED * (e % 8), ED)]
        pltpu.sync_copy(stg, accea.at[dst_v.at[j]], add=True)

    load(0, p0v, s0)

    @pl.loop(0, (NCHUNK - 1) // 2)
    def _(i):
        a = 2 * i
        load(a + 1, p1v, s1)
        drain(p0v, s0)
        unpack_scatter(a, p0v)
        load(a + 2, p0v, s0)
        drain(p1v, s1)
        unpack_scatter(a + 1, p1v)

    drain(p0v, s0)
    unpack_scatter(NCHUNK - 1, p0v)

    plsc.subcore_barrier()
    pltpu.sync_copy(accea.at[pl.ds(r0, RPT)],
                    outea_hbm.at[cid, pl.ds(r0, RPT)])


BR = 512


def _tc_layer_body(sx_ref, sea_ref, x_ref, We_ref, be_ref, Wl_ref, bl_ref,
                   Wr_ref, o_ref):
    sx = sx_ref[0] + sx_ref[1]
    sea = sea_ref[0] + sea_ref[1]
    cnt = sea[:, ED:ED + 1]
    inv = 1.0 / jnp.maximum(cnt, 1.0)
    has = jnp.where(cnt > 0.0, 1.0, 0.0)
    hi = jax.lax.Precision.HIGHEST
    agg = (sx * inv
           + jnp.dot(sea[:, :ED] * inv, We_ref[...],
                     preferred_element_type=_f32, precision=hi)
           + be_ref[...] * has)
    out = (jnp.dot(agg, Wl_ref[...], preferred_element_type=_f32, precision=hi)
           + bl_ref[...]
           + jnp.dot(x_ref[...], Wr_ref[...],
                     preferred_element_type=_f32, precision=hi))
    o_ref[...] = jnp.maximum(out, 0.0)


def _tc_layer(sx, sea, x, We, be, Wl, bl, Wr):
    grid = (NP // BR,)
    return pl.pallas_call(
        _tc_layer_body,
        grid=grid,
        in_specs=[
            pl.BlockSpec((NC, BR, D), lambda i: (0, i, 0)),
            pl.BlockSpec((NC, BR, D), lambda i: (0, i, 0)),
            pl.BlockSpec((BR, D), lambda i: (i, 0)),
            pl.BlockSpec((ED, D), lambda i: (0, 0)),
            pl.BlockSpec((1, D), lambda i: (0, 0)),
            pl.BlockSpec((D, D), lambda i: (0, 0)),
            pl.BlockSpec((1, D), lambda i: (0, 0)),
            pl.BlockSpec((D, D), lambda i: (0, 0)),
        ],
        out_specs=pl.BlockSpec((BR, D), lambda i: (i, 0)),
        out_shape=jax.ShapeDtypeStruct((NP, D), _f32),
    )(sx, sea, x, We, be.reshape(1, D), Wl, bl.reshape(1, D), Wr)


def kernel(x, edge_index, edge_attr, Wl0, bl0, Wr0, We0, be0,
           Wl1, bl1, Wr1, We1, be1):
    src = edge_index[0]
    dst = edge_index[1]
    pad = E_PAD - E
    src_p = jnp.concatenate([src, jnp.zeros((pad,), jnp.int32)])
    pad_dst = N + jnp.arange(pad, dtype=jnp.int32) % (NP - N)
    dst_p = jnp.concatenate([dst, pad_dst])
    srcs = src_p.reshape(NW, NCHUNK, CH)
    dsts = dst_p.reshape(NW, NCHUNK, CH)
    eap = edge_attr.reshape(E // 8, D)
    eap = jnp.concatenate([eap, jnp.zeros((pad // 8, D), _f32)], axis=0)
    eap = eap.reshape(NW, NCHUNK, PK, D)
    x_pad = jnp.concatenate([x, jnp.zeros((NP - N, D), _f32)], axis=0)
    zx = jnp.zeros((NP, D), _f32)

    sea = _sc_ea(eap, dsts, zx)
    sx0 = _sc_layer(x_pad, srcs, dsts, zx)
    x1 = _tc_layer(sx0, sea, x_pad, We0, be0, Wl0, bl0, Wr0)
    sx1 = _sc_layer(x1, srcs, dsts, zx)
    x2 = _tc_layer(sx1, sea, x1, We1, be1, Wl1, bl1, Wr1)
    return x2[:N]
